# baseline (speedup 1.0000x reference)
import jax
import jax.numpy as jnp
from jax import lax
from jax.experimental import pallas as pl
from jax.experimental.pallas import tpu as pltpu
from jax.experimental.pallas import tpu_sc as plsc

B = 16384
D = 16
NF = 26
NC = 2
NS = 16
NW = NC * NS
BPW = B // NW


def _body(*refs):
    idx_refs = refs[:NF]
    tbl_refs = refs[NF:2 * NF]
    out_ref = refs[2 * NF]
    idx_v, rows_v, sem = refs[2 * NF + 1:]

    wid = lax.axis_index("s") * NC + lax.axis_index("c")
    base = wid * BPW

    for f in range(NF):
        pltpu.sync_copy(idx_refs[f].at[pl.ds(base, BPW)], idx_v)
        pltpu.async_copy(tbl_refs[f].at[idx_v], rows_v, sem).wait()
        pltpu.sync_copy(rows_v, out_ref.at[pl.ds(base, BPW), pl.ds(f * D, D)])


def kernel(f0, f1, f2, f3, f4, f5, f6, f7, f8, f9, f10, f11, f12, f13, f14,
           f15, f16, f17, f18, f19, f20, f21, f22, f23, f24, f25,
           W_f0, W_f1, W_f2, W_f3, W_f4, W_f5, W_f6, W_f7, W_f8, W_f9,
           W_f10, W_f11, W_f12, W_f13, W_f14, W_f15, W_f16, W_f17, W_f18,
           W_f19, W_f20, W_f21, W_f22, W_f23, W_f24, W_f25):
    idx = (f0, f1, f2, f3, f4, f5, f6, f7, f8, f9, f10, f11, f12, f13, f14,
           f15, f16, f17, f18, f19, f20, f21, f22, f23, f24, f25)
    tbls = (W_f0, W_f1, W_f2, W_f3, W_f4, W_f5, W_f6, W_f7, W_f8, W_f9,
            W_f10, W_f11, W_f12, W_f13, W_f14, W_f15, W_f16, W_f17, W_f18,
            W_f19, W_f20, W_f21, W_f22, W_f23, W_f24, W_f25)

    mesh = plsc.VectorSubcoreMesh(core_axis_name="c", subcore_axis_name="s",
                                  num_cores=NC, num_subcores=NS)
    run = pl.kernel(
        _body,
        out_type=jax.ShapeDtypeStruct((B, NF * D), jnp.float32),
        mesh=mesh,
        scratch_types=[
            pltpu.VMEM((BPW,), jnp.int32),
            pltpu.VMEM((BPW, D), jnp.float32),
            pltpu.SemaphoreType.DMA,
        ],
        compiler_params=pltpu.CompilerParams(use_tc_tiling_on_sc=False),
    )
    return run(*idx, *tbls)


# speedup vs baseline: 1.0163x; 1.0163x over previous
import jax
import jax.numpy as jnp
from jax import lax
from jax.experimental import pallas as pl
from jax.experimental.pallas import tpu as pltpu
from jax.experimental.pallas import tpu_sc as plsc

B = 16384
D = 16
NF = 26
NC = 2
NS = 16
NW = NC * NS
BPW = B // NW
NBUF = 8


def _body(*refs):
    idx_refs = refs[:NF]
    tbl_refs = refs[NF:2 * NF]
    out_ref = refs[2 * NF]
    rest = refs[2 * NF + 1:]
    idx_v = rest[0]
    bufs = rest[1:1 + NBUF]
    sem_i = rest[1 + NBUF]
    sem_g = rest[2 + NBUF:2 + 2 * NBUF]
    sem_w = rest[2 + 2 * NBUF:2 + 3 * NBUF]

    wid = lax.axis_index("s") * NC + lax.axis_index("c")
    base = wid * BPW

    idx_copies = [
        pltpu.async_copy(idx_refs[f].at[pl.ds(base, BPW)], idx_v.at[f], sem_i)
        for f in range(NF)
    ]
    for c in idx_copies:
        c.wait()

    gathers = [None] * NF
    writes = [None] * NF
    for f in range(NBUF):
        gathers[f] = pltpu.async_copy(
            tbl_refs[f].at[idx_v.at[f]], bufs[f], sem_g[f])
    for f in range(NF):
        slot = f % NBUF
        gathers[f].wait()
        writes[f] = pltpu.async_copy(
            bufs[slot],
            out_ref.at[pl.ds(base, BPW), pl.ds(f * D, D)],
            sem_w[slot])
        g = f + NBUF
        if g < NF:
            writes[f].wait()
            gathers[g] = pltpu.async_copy(
                tbl_refs[g].at[idx_v.at[g]], bufs[slot], sem_g[slot])
    for f in range(NF - NBUF, NF):
        writes[f].wait()


def kernel(f0, f1, f2, f3, f4, f5, f6, f7, f8, f9, f10, f11, f12, f13, f14,
           f15, f16, f17, f18, f19, f20, f21, f22, f23, f24, f25,
           W_f0, W_f1, W_f2, W_f3, W_f4, W_f5, W_f6, W_f7, W_f8, W_f9,
           W_f10, W_f11, W_f12, W_f13, W_f14, W_f15, W_f16, W_f17, W_f18,
           W_f19, W_f20, W_f21, W_f22, W_f23, W_f24, W_f25):
    idx = (f0, f1, f2, f3, f4, f5, f6, f7, f8, f9, f10, f11, f12, f13, f14,
           f15, f16, f17, f18, f19, f20, f21, f22, f23, f24, f25)
    tbls = (W_f0, W_f1, W_f2, W_f3, W_f4, W_f5, W_f6, W_f7, W_f8, W_f9,
            W_f10, W_f11, W_f12, W_f13, W_f14, W_f15, W_f16, W_f17, W_f18,
            W_f19, W_f20, W_f21, W_f22, W_f23, W_f24, W_f25)

    mesh = plsc.VectorSubcoreMesh(core_axis_name="c", subcore_axis_name="s",
                                  num_cores=NC, num_subcores=NS)
    run = pl.kernel(
        _body,
        out_type=jax.ShapeDtypeStruct((B, NF * D), jnp.float32),
        mesh=mesh,
        scratch_types=(
            [pltpu.VMEM((NF, BPW), jnp.int32)]
            + [pltpu.VMEM((BPW, D), jnp.float32) for _ in range(NBUF)]
            + [pltpu.SemaphoreType.DMA]
            + [pltpu.SemaphoreType.DMA for _ in range(2 * NBUF)]
        ),
        compiler_params=pltpu.CompilerParams(use_tc_tiling_on_sc=False),
    )
    return run(*idx, *tbls)
